# baseline (speedup 1.0000x reference)
import jax
import jax.numpy as jnp
from jax import lax
from jax.experimental import pallas as pl
from jax.experimental.pallas import tpu as pltpu

NUM_GRAPHS = 64
NUM_VIRTUAL = 4
N_NODES = 10000
HIDDEN = 128


def _fused_kernel(x_ref, batch_ref, W_enc_ref, b_enc_ref, W1_ref, b1_ref,
                  W2_ref, b2_ref, Wp1_ref, bp1_ref, Wp2_ref, bp2_ref,
                  out_ref):
    nf = jnp.maximum(jnp.dot(x_ref[...], W_enc_ref[...]) + b_enc_ref[...],
                     0.0)
    bb = batch_ref[0, :]
    onehot_t = (lax.broadcasted_iota(jnp.int32, (NUM_GRAPHS, N_NODES), 0)
                == bb[None, :]).astype(jnp.float32)
    seg = jnp.dot(onehot_t, nf) * (1.0 / NUM_VIRTUAL)
    h = jnp.maximum(jnp.dot(seg, W1_ref[...]) + b1_ref[...], 0.0)
    gf = jnp.dot(h, W2_ref[...]) + b2_ref[...]
    p = jnp.maximum(jnp.dot(gf, Wp1_ref[...]) + bp1_ref[...], 0.0)
    out_ref[...] = jnp.dot(p, Wp2_ref[...]) + bp2_ref[...]


def kernel(x, edge_index, batch, W_enc, b_enc, W1, b1, W2, b2, Wp1, bp1,
           Wp2, bp2):
    del edge_index
    vmem = pl.BlockSpec(memory_space=pltpu.MemorySpace.VMEM)
    out = pl.pallas_call(
        _fused_kernel,
        in_specs=[vmem] * 12,
        out_specs=vmem,
        out_shape=jax.ShapeDtypeStruct((NUM_GRAPHS, 1), jnp.float32),
    )(x, batch.reshape(1, N_NODES), W_enc, b_enc.reshape(1, HIDDEN),
      W1, b1.reshape(1, HIDDEN), W2, b2.reshape(1, HIDDEN),
      Wp1, bp1.reshape(1, HIDDEN), Wp2, bp2.reshape(1, 1))
    return out
